# baseline (speedup 1.0000x reference)
import functools

import jax
import jax.numpy as jnp
from jax import lax
from jax.experimental import pallas as pl
from jax.experimental.pallas import tpu as pltpu
from jax.experimental.pallas import tpu_sc as plsc

N = 10000
NP = 10240
E = 320000
EP = 323584
HH = 128
H = 256
G = 64
R = 512
C1 = 64
EW1 = EP // 32
NCH1 = EW1 // C1
C2 = 64
EW2 = EP // 16
EW2H = EW2 // 2
NCHP = EW2H // C2
RW = NP // 32
PC = 80
DR = NP // 128
NEG = -3.0e38


def _k1_body(x_ref, Wl_ref, bl_ref, Wr_ref, br_ref, We_ref, att_ref, ea_ref,
             axl_ref, axr_ref, xlA_ref, xlB_ref, selfw_ref, mea_ref, mea_smem):
    i = pl.program_id(0)

    @pl.when(i == 0)
    def _():
        sv = jnp.sum(ea_ref[...], axis=0)
        par = lax.broadcasted_iota(jnp.int32, (128,), 0) % 2
        mea_smem[0] = jnp.sum(jnp.where(par == 0, sv, 0.0)) / E
        mea_smem[1] = jnp.sum(jnp.where(par == 1, sv, 0.0)) / E

    h = jnp.maximum(x_ref[...], 0.0)
    att = att_ref[...]
    xl = jnp.dot(h, Wl_ref[...], preferred_element_type=jnp.float32) + bl_ref[...]
    xr = jnp.dot(h, Wr_ref[...], preferred_element_type=jnp.float32) + br_ref[...]
    axl = att[None, :] * xl
    axr = att[None, :] * xr
    aWe0 = att * We_ref[0, :]
    aWe1 = att * We_ref[1, :]
    s = jnp.sign(att)
    cself = mea_smem[0] * aWe0 + mea_smem[1] * aWe1
    u = axl + axr + cself[None, :]
    logit = 0.6 * jnp.sum(u, axis=1) + 0.4 * jnp.sum(s[None, :] * jnp.abs(u), axis=1)
    axl_ref[...] = axl
    axr_ref[...] = axr
    xlA_ref[...] = xl[:, :HH]
    xlB_ref[...] = xl[:, HH:]
    selfw_ref[...] = jnp.exp(logit)
    mea_ref[...] = jnp.concatenate(
        [jnp.full((1, 128), mea_smem[0], jnp.float32),
         jnp.full((1, 128), mea_smem[1], jnp.float32)], axis=0)


def _k1(x_pad, W1l, b1l, W1r, b1r, We1, att1, ea_rs):
    grid = (NP // R,)
    full2 = lambda shp: pl.BlockSpec(shp, lambda i: (0,) * len(shp))
    return pl.pallas_call(
        _k1_body,
        grid=grid,
        in_specs=[
            pl.BlockSpec((R, 128), lambda i: (i, 0)),
            full2((128, H)), full2((H,)), full2((128, H)), full2((H,)),
            full2((2, H)), full2((H,)), full2((5000, 128)),
        ],
        out_specs=[
            pl.BlockSpec((R, H), lambda i: (i, 0)),
            pl.BlockSpec((R, H), lambda i: (i, 0)),
            pl.BlockSpec((R, HH), lambda i: (i, 0)),
            pl.BlockSpec((R, HH), lambda i: (i, 0)),
            pl.BlockSpec((R,), lambda i: (i,)),
            pl.BlockSpec((2, 128), lambda i: (0, 0)),
        ],
        out_shape=[
            jax.ShapeDtypeStruct((NP, H), jnp.float32),
            jax.ShapeDtypeStruct((NP, H), jnp.float32),
            jax.ShapeDtypeStruct((NP, HH), jnp.float32),
            jax.ShapeDtypeStruct((NP, HH), jnp.float32),
            jax.ShapeDtypeStruct((NP,), jnp.float32),
            jax.ShapeDtypeStruct((2, 128), jnp.float32),
        ],
        scratch_shapes=[pltpu.SMEM((2,), jnp.float32)],
    )(x_pad, W1l, b1l, W1r, b1r, We1, att1, ea_rs)


def _k4_body(accA_ref, accB_ref, denp_ref, selfw_ref,
             xlA_ref, xlB_ref, bias_ref,
             Wl_ref, bl_ref, Wr_ref, br_ref, We_ref, att_ref, mea_ref,
             axl_ref, axr_ref, xlA2_ref, xlB2_ref, selfw2_ref):
    selfw = selfw_ref[...]
    den = denp_ref[0, :] + denp_ref[1, :] + selfw
    inv = 1.0 / den
    hA = jnp.maximum(
        (accA_ref[...] + selfw[:, None] * xlA_ref[...]) * inv[:, None]
        + bias_ref[:HH][None, :], 0.0)
    hB = jnp.maximum(
        (accB_ref[...] + selfw[:, None] * xlB_ref[...]) * inv[:, None]
        + bias_ref[HH:][None, :], 0.0)
    Wl = Wl_ref[...]
    Wr = Wr_ref[...]
    xl = (jnp.dot(hA, Wl[:HH, :], preferred_element_type=jnp.float32)
          + jnp.dot(hB, Wl[HH:, :], preferred_element_type=jnp.float32) + bl_ref[...])
    xr = (jnp.dot(hA, Wr[:HH, :], preferred_element_type=jnp.float32)
          + jnp.dot(hB, Wr[HH:, :], preferred_element_type=jnp.float32) + br_ref[...])
    att = att_ref[...]
    axl = att[None, :] * xl
    axr = att[None, :] * xr
    aWe0 = att * We_ref[0, :]
    aWe1 = att * We_ref[1, :]
    s = jnp.sign(att)
    cself = mea_ref[0, 0] * aWe0 + mea_ref[1, 0] * aWe1
    u = axl + axr + cself[None, :]
    logit = 0.6 * jnp.sum(u, axis=1) + 0.4 * jnp.sum(s[None, :] * jnp.abs(u), axis=1)
    axl_ref[...] = axl
    axr_ref[...] = axr
    xlA2_ref[...] = xl[:, :HH]
    xlB2_ref[...] = xl[:, HH:]
    selfw2_ref[...] = jnp.exp(logit)


def _k4(accA, accB, denp, selfw, xlA, xlB, bias,
        Wl, bl, Wr, br, We, att, mea):
    grid = (NP // R,)
    full2 = lambda shp: pl.BlockSpec(shp, lambda i: (0,) * len(shp))
    return pl.pallas_call(
        _k4_body,
        grid=grid,
        in_specs=[
            pl.BlockSpec((R, HH), lambda i: (i, 0)),
            pl.BlockSpec((R, HH), lambda i: (i, 0)),
            pl.BlockSpec((2, R), lambda i: (0, i)),
            pl.BlockSpec((R,), lambda i: (i,)),
            pl.BlockSpec((R, HH), lambda i: (i, 0)),
            pl.BlockSpec((R, HH), lambda i: (i, 0)),
            full2((H,)),
            full2((H, H)), full2((H,)), full2((H, H)), full2((H,)),
            full2((2, H)), full2((H,)), full2((2, 128)),
        ],
        out_specs=[
            pl.BlockSpec((R, H), lambda i: (i, 0)),
            pl.BlockSpec((R, H), lambda i: (i, 0)),
            pl.BlockSpec((R, HH), lambda i: (i, 0)),
            pl.BlockSpec((R, HH), lambda i: (i, 0)),
            pl.BlockSpec((R,), lambda i: (i,)),
        ],
        out_shape=[
            jax.ShapeDtypeStruct((NP, H), jnp.float32),
            jax.ShapeDtypeStruct((NP, H), jnp.float32),
            jax.ShapeDtypeStruct((NP, HH), jnp.float32),
            jax.ShapeDtypeStruct((NP, HH), jnp.float32),
            jax.ShapeDtypeStruct((NP,), jnp.float32),
        ],
    )(accA, accB, denp, selfw, xlA, xlB, bias,
      Wl, bl, Wr, br, We, att, mea)


def _pass1_body(axl_h, axr_h, att_h, We_h, src_h, dst_h, ea0_h, ea1_h,
                w_h, denp_h,
                attv, wev, aWe0, aWe1, sv, srcL, dstL, wloc,
                hbufA, hbufB, rowvA, rowvB,
                bufLA, bufRA, bufLB, bufRB, ea0A, ea1A, ea0B, ea1B,
                shden, semA, semB, semHA, semHB):
    c = lax.axis_index("c")
    s = lax.axis_index("s")
    wid = s * 2 + c
    base0 = wid * EW1
    pltpu.sync_copy(att_h, attv)
    pltpu.sync_copy(We_h, wev)
    pltpu.sync_copy(src_h.at[pl.ds(base0, EW1)], srcL)
    pltpu.sync_copy(dst_h.at[pl.ds(base0, EW1)], dstL)
    for j in range(16):
        sl = pl.ds(j * 16, 16)
        a = attv[sl]
        aWe0[sl] = a * wev[0, sl]
        aWe1[sl] = a * wev[1, sl]
        sv[sl] = jnp.sign(a)

    lane_iota = lax.iota(jnp.int32, 16)
    zero16 = jnp.zeros((16,), jnp.float32)
    rot_idx = [(lane_iota + sh) & 15 for sh in (8, 4, 2, 1)]

    def rsum(v):
        for idx in rot_idx:
            v = v + v[idx]
        return v

    izero = jnp.zeros((16,), jnp.int32)

    def zrow(r, carry):
        for j in range(8):
            hbufA[r, pl.ds(j * 16, 16)] = zero16
            hbufB[r, pl.ds(j * 16, 16)] = zero16
        return carry

    lax.fori_loop(0, C1, zrow, 0, unroll=False)
    for j in range(C1 // 16):
        sl = pl.ds(j * 16, 16)
        rowvA[sl] = izero
        rowvB[sl] = izero
    pltpu.sync_copy(hbufA.at[pl.ds(0, DR // 16)],
                    shden.at[pl.ds(s * (DR // 16), DR // 16)])
    plsc.subcore_barrier()

    bufs = {0: (bufLA, bufRA, ea0A, ea1A, semA, hbufA, rowvA, semHA),
            1: (bufLB, bufRB, ea0B, ea1B, semB, hbufB, rowvB, semHB)}

    def fire_hist(par):
        bL, bR, e0b, e1b, sem, hb, rv, semH = bufs[par]
        pltpu.async_copy(hb, shden.at[rv], semH, add=True)

    def drain_hist(par):
        bL, bR, e0b, e1b, sem, hb, rv, semH = bufs[par]
        pltpu.make_async_copy(
            axl_h.at[pl.ds(0, C1), pl.ds(0, 128)], hb, semH).wait()

    def issue(cidx, par):
        bL, bR, e0b, e1b, sem = bufs[par][:5]
        pltpu.async_copy(axl_h.at[srcL.at[pl.ds(cidx * C1, C1)]], bL, sem)
        pltpu.async_copy(axr_h.at[dstL.at[pl.ds(cidx * C1, C1)]], bR, sem)
        pltpu.async_copy(ea0_h.at[pl.ds(base0 + cidx * C1, C1)], e0b, sem)
        pltpu.async_copy(ea1_h.at[pl.ds(base0 + cidx * C1, C1)], e1b, sem)

    def drain(par):
        bL, bR, e0b, e1b, sem = bufs[par][:5]
        pltpu.make_async_copy(axl_h.at[pl.ds(0, C1)], bL, sem).wait()
        pltpu.make_async_copy(axl_h.at[pl.ds(0, C1)], bR, sem).wait()
        pltpu.make_async_copy(ea0_h.at[pl.ds(0, C1)], e0b, sem).wait()
        pltpu.make_async_copy(ea0_h.at[pl.ds(0, C1)], e1b, sem).wait()

    def compute(cidx, par):
        bL, bR, e0b, e1b, sem, hbuf, rowv, semH = bufs[par]
        drain_hist(par)
        aWe0r = [aWe0[pl.ds(16 * j, 16)] for j in range(16)]
        aWe1r = [aWe1[pl.ds(16 * j, 16)] for j in range(16)]
        svr = [sv[pl.ds(16 * j, 16)] for j in range(16)]

        def edge(el, wcar):
            gb16 = jnp.bitwise_and(el, -16)
            lane = jnp.bitwise_and(el, 15)
            spl = jnp.full((16,), lane, jnp.int32)
            e0v = e0b[pl.ds(gb16, 16)][spl]
            e1v = e1b[pl.ds(gb16, 16)][spl]
            lin = zero16
            ab = zero16
            for j in range(16):
                sl = pl.ds(j * 16, 16)
                u = bufs[par][0][el, sl] + bufs[par][1][el, sl] \
                    + e0v * aWe0r[j] + e1v * aWe1r[j]
                lin = lin + u
                ab = ab + svr[j] * jnp.abs(u)
            lv = rsum(0.6 * lin + 0.4 * ab)
            dvec = dstL[pl.ds(cidx * C1 + gb16, 16)][spl]
            colv = jnp.bitwise_and(dvec, 127)
            expv = jnp.exp(lv)
            for j in range(8):
                hbuf[el, pl.ds(j * 16, 16)] = jnp.where(
                    lane_iota + (j * 16) == colv, expv, 0.0)
            wcar = jnp.where(lane_iota == (el & 15), lv, wcar)

            @pl.when((el & 15) == 15)
            def _():
                wloc[pl.ds(cidx * C1 + el - 15, 16)] = jnp.exp(wcar)

            return wcar

        def edge2(t, wcar):
            wcar = edge(2 * t, wcar)
            return edge(2 * t + 1, wcar)

        lax.fori_loop(0, C1 // 2, edge2, zero16, unroll=False)
        for j in range(C1 // 16):
            sl = pl.ds(j * 16, 16)
            rowv[sl] = jnp.right_shift(dstL[pl.ds(cidx * C1 + j * 16, 16)], 7)
        fire_hist(par)

    fire_hist(0)
    fire_hist(1)
    issue(0, 0)

    def pair(p, carry):
        issue(2 * p + 1, 1)
        drain(0)
        compute(2 * p, 0)
        issue(2 * p + 2, 0)
        drain(1)
        compute(2 * p + 1, 1)
        return carry

    lax.fori_loop(0, NCH1 // 2 - 1, pair, 0, unroll=False)
    issue(NCH1 - 1, 1)
    drain(0)
    compute(NCH1 - 2, 0)
    drain(1)
    compute(NCH1 - 1, 1)
    drain_hist(0)
    drain_hist(1)

    pltpu.sync_copy(wloc, w_h.at[pl.ds(base0, EW1)])
    plsc.subcore_barrier()

    @pl.when(s == 0)
    def _():
        pltpu.sync_copy(shden, denp_h.at[pl.ds(c * DR, DR)])


def _pass1(axl, axr, att, We, src, dst, ea0, ea1):
    mesh = plsc.VectorSubcoreMesh(core_axis_name="c", subcore_axis_name="s")
    f = functools.partial(
        pl.kernel,
        out_type=[
            jax.ShapeDtypeStruct((EP,), jnp.float32),
            jax.ShapeDtypeStruct((2 * DR, 128), jnp.float32),
        ],
        mesh=mesh,
        scratch_types=[
            pltpu.VMEM((H,), jnp.float32),
            pltpu.VMEM((2, H), jnp.float32),
            pltpu.VMEM((H,), jnp.float32),
            pltpu.VMEM((H,), jnp.float32),
            pltpu.VMEM((H,), jnp.float32),
            pltpu.VMEM((EW1,), jnp.int32),
            pltpu.VMEM((EW1,), jnp.int32),
            pltpu.VMEM((EW1,), jnp.float32),
            pltpu.VMEM((C1, 128), jnp.float32),
            pltpu.VMEM((C1, 128), jnp.float32),
            pltpu.VMEM((C1,), jnp.int32),
            pltpu.VMEM((C1,), jnp.int32),
            pltpu.VMEM((C1, H), jnp.float32),
            pltpu.VMEM((C1, H), jnp.float32),
            pltpu.VMEM((C1, H), jnp.float32),
            pltpu.VMEM((C1, H), jnp.float32),
            pltpu.VMEM((C1,), jnp.float32),
            pltpu.VMEM((C1,), jnp.float32),
            pltpu.VMEM((C1,), jnp.float32),
            pltpu.VMEM((C1,), jnp.float32),
            pltpu.VMEM_SHARED((DR, 128), jnp.float32),
            pltpu.SemaphoreType.DMA,
            pltpu.SemaphoreType.DMA,
            pltpu.SemaphoreType.DMA,
            pltpu.SemaphoreType.DMA,
        ],
    )(_pass1_body)
    return f(axl, axr, att, We, src, dst, ea0, ea1)


def _kden_body(denp_ref, den_ref):
    den_ref[...] = denp_ref[0, :] + denp_ref[1, :]


def _kden(denp):
    return pl.pallas_call(
        _kden_body,
        grid=(NP // R,),
        in_specs=[pl.BlockSpec((2, R), lambda i: (0, i))],
        out_specs=pl.BlockSpec((R,), lambda i: (i,)),
        out_shape=jax.ShapeDtypeStruct((NP,), jnp.float32),
    )(denp)


def _pass2_body(xlF_h, srcB_h, dst_h, w_h,
                acc_h,
                shacc, srcLoc, dstvA, dstvB, dscA, dscB, wvA, wvB,
                gbufA, gbufB, sbufA, sbufB,
                semA, semB, semSA, semSB):
    c = lax.axis_index("c")
    s = lax.axis_index("s")
    zero16 = jnp.zeros((16,), jnp.float32)
    izero = jnp.zeros((16,), jnp.int32)

    def zrow(r, carry):
        for j in range(8):
            sbufA[r, pl.ds(j * 16, 16)] = zero16
            sbufB[r, pl.ds(j * 16, 16)] = zero16
        return carry

    lax.fori_loop(0, C2, zrow, 0, unroll=False)
    for t in range(10):
        pltpu.sync_copy(sbufA, shacc.at[pl.ds(s * 640 + t * 64, 64)])
    for j in range(C2 // 16):
        sl = pl.ds(j * 16, 16)
        dstvA[sl] = izero
        dstvB[sl] = izero
        dscA[sl] = izero
        dscB[sl] = izero
    plsc.subcore_barrier()

    base0 = s * EW2
    bufs = {0: (gbufA, dstvA, dscA, wvA, sbufA, semA, semSA),
            1: (gbufB, dstvB, dscB, wvB, sbufB, semB, semSB)}

    def drain(par):
        gb, dv, dc, wv, sb, sem, semS = bufs[par]
        pltpu.make_async_copy(xlF_h.at[pl.ds(0, C2)], gb, sem).wait()
        pltpu.make_async_copy(dst_h.at[pl.ds(0, C2)], dv, sem).wait()
        pltpu.make_async_copy(w_h.at[pl.ds(0, C2)], wv, sem).wait()

    def drain_scatter(par):
        gb, dv, dc, wv, sb, sem, semS = bufs[par]
        pltpu.make_async_copy(xlF_h.at[pl.ds(0, C2)], sb, semS).wait()

    def fire_scatter(par):
        gb, dv, dc, wv, sb, sem, semS = bufs[par]
        pltpu.async_copy(sb, shacc.at[dc], semS, add=True)

    def compute(cidx_unused, par):
        gb, dv, dc, wv, sb, sem, semS = bufs[par]
        for j in range(C2 // 16):
            sl = pl.ds(j * 16, 16)
            dc[sl] = dv[sl]

        def group(g, carry2):
            wg = wv[pl.ds(g * 16, 16)]
            for i in range(16):
                e = g * 16 + i
                we = wg[i]
                for j in range(8):
                    sl = pl.ds(j * 16, 16)
                    sb[e, sl] = we * gb[e, sl]
            return carry2

        lax.fori_loop(0, C2 // 16, group, 0, unroll=False)

    fire_scatter(0)
    fire_scatter(1)

    def phase(ph, carry):
        pbase = base0 + ph * EW2H
        pltpu.sync_copy(srcB_h.at[pl.ds(c * EP + pbase, EW2H)], srcLoc)

        def issue(cidx, par):
            gb, dv, dc, wv, sb, sem, semS = bufs[par]
            pltpu.async_copy(
                xlF_h.at[srcLoc.at[pl.ds(cidx * C2, C2)]], gb, sem)
            pltpu.async_copy(dst_h.at[pl.ds(pbase + cidx * C2, C2)], dv, sem)
            pltpu.async_copy(w_h.at[pl.ds(pbase + cidx * C2, C2)], wv, sem)

        issue(0, 0)

        def pair(p, carry2):
            issue(2 * p + 1, 1)
            drain(0)
            drain_scatter(0)
            compute(2 * p, 0)
            fire_scatter(0)
            issue(2 * p + 2, 0)
            drain(1)
            drain_scatter(1)
            compute(2 * p + 1, 1)
            fire_scatter(1)
            return carry2

        lax.fori_loop(0, NCHP // 2 - 1, pair, 0, unroll=False)
        issue(NCHP - 1, 1)
        drain(0)
        drain_scatter(0)
        compute(NCHP - 2, 0)
        fire_scatter(0)
        drain(1)
        drain_scatter(1)
        compute(NCHP - 1, 1)
        fire_scatter(1)
        return carry

    lax.fori_loop(0, 2, phase, 0, unroll=False)
    drain_scatter(0)
    drain_scatter(1)
    plsc.subcore_barrier()
    pltpu.sync_copy(shacc.at[pl.ds(s * 640, 640)],
                    acc_h.at[pl.ds(c * NP + s * 640, 640)])


def _pass2(xl_flat, srcB, dst, w):
    mesh = plsc.VectorSubcoreMesh(core_axis_name="c", subcore_axis_name="s")
    f = functools.partial(
        pl.kernel,
        out_type=jax.ShapeDtypeStruct((2 * NP, HH), jnp.float32),
        mesh=mesh,
        scratch_types=[
            pltpu.VMEM_SHARED((NP, HH), jnp.float32),
            pltpu.VMEM((EW2H,), jnp.int32),
            pltpu.VMEM((C2,), jnp.int32),
            pltpu.VMEM((C2,), jnp.int32),
            pltpu.VMEM((C2,), jnp.int32),
            pltpu.VMEM((C2,), jnp.int32),
            pltpu.VMEM((C2,), jnp.float32),
            pltpu.VMEM((C2,), jnp.float32),
            pltpu.VMEM((C2, HH), jnp.float32),
            pltpu.VMEM((C2, HH), jnp.float32),
            pltpu.VMEM((C2, HH), jnp.float32),
            pltpu.VMEM((C2, HH), jnp.float32),
            pltpu.SemaphoreType.DMA,
            pltpu.SemaphoreType.DMA,
            pltpu.SemaphoreType.DMA,
            pltpu.SemaphoreType.DMA,
        ],
    )(_pass2_body)
    return f(xl_flat, srcB, dst, w)


def _pool_body(accA_h, accB_h, den_h, selfw_h, xlA_h, xlB_h,
               bias_h, batch_h,
               psum_h, pmax_h, pcnt_h,
               biasv, batv, swv, d0v, bA, bB, bxA, bxB,
               gsum, gmax, gcnt):
    c = lax.axis_index("c")
    s = lax.axis_index("s")
    wid = s * 2 + c
    lane_iota = lax.iota(jnp.int32, 16)
    lane0 = jnp.where(lane_iota == 0, 1.0, 0.0)
    pltpu.sync_copy(bias_h, biasv)

    def zrow(g, carry):
        for j in range(16):
            sl = pl.ds(j * 16, 16)
            gsum[g, sl] = jnp.zeros((16,), jnp.float32)
            gmax[g, sl] = jnp.full((16,), NEG, jnp.float32)
        gcnt[g, pl.ds(0, 16)] = jnp.zeros((16,), jnp.float32)
        return carry

    lax.fori_loop(0, G, zrow, 0, unroll=False)

    def chunkfn(t, carry):
        base = wid * RW + t * PC
        pltpu.sync_copy(accA_h.at[pl.ds(base, PC)], bA)
        pltpu.sync_copy(accB_h.at[pl.ds(base, PC)], bB)
        pltpu.sync_copy(den_h.at[pl.ds(base, PC)], d0v)
        pltpu.sync_copy(selfw_h.at[pl.ds(base, PC)], swv)
        pltpu.sync_copy(xlA_h.at[pl.ds(base, PC)], bxA)
        pltpu.sync_copy(xlB_h.at[pl.ds(base, PC)], bxB)
        pltpu.sync_copy(batch_h.at[pl.ds(base, PC)], batv)

        def rowgroup(gg, carry2):
            batg = batv[pl.ds(gg * 16, 16)]
            swg = swv[pl.ds(gg * 16, 16)]
            d0g = d0v[pl.ds(gg * 16, 16)]
            invg = 1.0 / (d0g + swg)
            for i in range(16):
                r = gg * 16 + i
                b = batg[i]
                sw = swg[i]
                inv = invg[i]

                @pl.when(b < G)
                def _():
                    for j in range(8):
                        sl = pl.ds(j * 16, 16)
                        slB = pl.ds(HH + j * 16, 16)
                        hA = jnp.maximum(
                            (bA[r, sl] + sw * bxA[r, sl]) * inv + biasv[sl], 0.0)
                        hB = jnp.maximum(
                            (bB[r, sl] + sw * bxB[r, sl]) * inv + biasv[slB], 0.0)
                        gsum[b, sl] = gsum[b, sl] + hA
                        gsum[b, slB] = gsum[b, slB] + hB
                        gmax[b, sl] = jnp.maximum(gmax[b, sl], hA)
                        gmax[b, slB] = jnp.maximum(gmax[b, slB], hB)
                    gcnt[b, pl.ds(0, 16)] = gcnt[b, pl.ds(0, 16)] + lane0

            return carry2

        lax.fori_loop(0, PC // 16, rowgroup, 0, unroll=False)
        return carry

    lax.fori_loop(0, RW // PC, chunkfn, 0, unroll=False)
    pltpu.sync_copy(gsum, psum_h.at[pl.ds(wid * G, G)])
    pltpu.sync_copy(gmax, pmax_h.at[pl.ds(wid * G, G)])
    pltpu.sync_copy(gcnt, pcnt_h.at[pl.ds(wid * G, G)])


def _pool(accA, accB, den, selfw, xlA, xlB, bias, batch_pad):
    mesh = plsc.VectorSubcoreMesh(core_axis_name="c", subcore_axis_name="s")
    f = functools.partial(
        pl.kernel,
        out_type=[
            jax.ShapeDtypeStruct((32 * G, H), jnp.float32),
            jax.ShapeDtypeStruct((32 * G, H), jnp.float32),
            jax.ShapeDtypeStruct((32 * G, 16), jnp.float32),
        ],
        mesh=mesh,
        scratch_types=[
            pltpu.VMEM((H,), jnp.float32),
            pltpu.VMEM((PC,), jnp.int32),
            pltpu.VMEM((PC,), jnp.float32),
            pltpu.VMEM((PC,), jnp.float32),
            pltpu.VMEM((PC, HH), jnp.float32),
            pltpu.VMEM((PC, HH), jnp.float32),
            pltpu.VMEM((PC, HH), jnp.float32),
            pltpu.VMEM((PC, HH), jnp.float32),
            pltpu.VMEM((G, H), jnp.float32),
            pltpu.VMEM((G, H), jnp.float32),
            pltpu.VMEM((G, 16), jnp.float32),
        ],
    )(_pool_body)
    return f(accA, accB, den, selfw, xlA, xlB, bias, batch_pad)


def _k9_body(psum_ref, pmax_ref, pcnt_ref, fcW_ref, fcb_ref, out_ref):
    gsum = jnp.sum(psum_ref[...], axis=0)
    gmax = jnp.max(pmax_ref[...], axis=0)
    counts = jnp.sum(pcnt_ref[...], axis=(0, 2))
    gmean = gsum / jnp.maximum(counts, 1.0)[:, None]
    gmaxz = jnp.where(counts[:, None] > 0, gmax, 0.0)
    fcW = fcW_ref[...]
    out = (jnp.dot(gmaxz, fcW[:H, :], preferred_element_type=jnp.float32)
           + jnp.dot(gmean, fcW[H:, :], preferred_element_type=jnp.float32)
           + fcb_ref[0])
    out_ref[...] = out


def _k9(psum, pmax, pcnt, fcW, fcb):
    return pl.pallas_call(
        _k9_body,
        out_shape=jax.ShapeDtypeStruct((G, 1), jnp.float32),
    )(psum, pmax, pcnt, fcW, fcb)


def kernel(x, edge_attr, W1l, b1l, W1r, b1r, We1, att1, bias1,
           W2l, b2l, W2r, b2r, We2, att2, bias2, fcW, fcb,
           edge_index, batch):
    x_pad = jnp.pad(x, ((0, NP - N), (0, 0)))
    src = jnp.concatenate([edge_index[0], jnp.full((EP - E,), N, jnp.int32)])
    dst = jnp.concatenate([edge_index[1], jnp.full((EP - E,), N, jnp.int32)])
    ea0 = jnp.concatenate([edge_attr[:, 0], jnp.zeros((EP - E,), jnp.float32)])
    ea1 = jnp.concatenate([edge_attr[:, 1], jnp.zeros((EP - E,), jnp.float32)])
    ea_rs = jnp.reshape(edge_attr, (5000, 128))
    batch_pad = jnp.concatenate([batch, jnp.full((NP - N,), G, jnp.int32)])

    axl1, axr1, xlA1, xlB1, selfw1, mea = _k1(
        x_pad, W1l, b1l, W1r, b1r, We1, att1, ea_rs)
    w1, denp1 = _pass1(axl1, axr1, att1, We1, src, dst, ea0, ea1)
    denp1 = jnp.reshape(denp1, (2, NP))
    srcB = jnp.concatenate([src, src + NP])
    xl1_flat = jnp.concatenate([xlA1, xlB1], axis=0)
    acc1 = _pass2(xl1_flat, srcB, dst, w1)

    axl2, axr2, xlA2, xlB2, selfw2 = _k4(
        acc1[:NP], acc1[NP:], denp1, selfw1, xlA1, xlB1, bias1,
        W2l, b2l, W2r, b2r, We2, att2, mea)
    w2, denp2 = _pass1(axl2, axr2, att2, We2, src, dst, ea0, ea1)
    denp2 = jnp.reshape(denp2, (2, NP))
    xl2_flat = jnp.concatenate([xlA2, xlB2], axis=0)
    acc2 = _pass2(xl2_flat, srcB, dst, w2)
    den2 = _kden(denp2)

    psum, pmax, pcnt = _pool(
        acc2[:NP], acc2[NP:], den2, selfw2, xlA2, xlB2,
        bias2, batch_pad)
    psum = jnp.reshape(psum, (32, G, H))
    pmax = jnp.reshape(pmax, (32, G, H))
    pcnt = jnp.reshape(pcnt, (32, G, 16))
    return _k9(psum, pmax, pcnt, fcW, fcb)


# speedup vs baseline: 1.0688x; 1.0688x over previous
import functools

import jax
import jax.numpy as jnp
from jax import lax
from jax.experimental import pallas as pl
from jax.experimental.pallas import tpu as pltpu
from jax.experimental.pallas import tpu_sc as plsc

N = 10000
NP = 10240
E = 320000
EP = 323584
HH = 128
H = 256
G = 64
R = 512
C1 = 64
EW1 = EP // 32
NCH1 = EW1 // C1
C2 = 64
EW2 = EP // 16
EW2H = EW2 // 2
NCHP = EW2H // C2
RW = NP // 32
PC = 80
DR = NP // 128
NEG = -3.0e38


def _k1_body(x_ref, Wl_ref, bl_ref, Wr_ref, br_ref, We_ref, att_ref, ea_ref,
             axl_ref, axr_ref, xlA_ref, xlB_ref, selfw_ref, mea_ref, mea_smem):
    i = pl.program_id(0)

    @pl.when(i == 0)
    def _():
        sv = jnp.sum(ea_ref[...], axis=0)
        par = lax.broadcasted_iota(jnp.int32, (128,), 0) % 2
        mea_smem[0] = jnp.sum(jnp.where(par == 0, sv, 0.0)) / E
        mea_smem[1] = jnp.sum(jnp.where(par == 1, sv, 0.0)) / E

    h = jnp.maximum(x_ref[...], 0.0)
    att = att_ref[...]
    xl = jnp.dot(h, Wl_ref[...], preferred_element_type=jnp.float32) + bl_ref[...]
    xr = jnp.dot(h, Wr_ref[...], preferred_element_type=jnp.float32) + br_ref[...]
    axl = att[None, :] * xl
    axr = att[None, :] * xr
    aWe0 = att * We_ref[0, :]
    aWe1 = att * We_ref[1, :]
    s = jnp.sign(att)
    cself = mea_smem[0] * aWe0 + mea_smem[1] * aWe1
    u = axl + axr + cself[None, :]
    logit = 0.6 * jnp.sum(u, axis=1) + 0.4 * jnp.sum(s[None, :] * jnp.abs(u), axis=1)
    axl_ref[...] = axl
    axr_ref[...] = axr
    xlA_ref[...] = xl[:, :HH]
    xlB_ref[...] = xl[:, HH:]
    selfw_ref[...] = jnp.exp(logit)
    mea_ref[...] = jnp.concatenate(
        [jnp.full((1, 128), mea_smem[0], jnp.float32),
         jnp.full((1, 128), mea_smem[1], jnp.float32)], axis=0)


def _k1(x_pad, W1l, b1l, W1r, b1r, We1, att1, ea_rs):
    grid = (NP // R,)
    full2 = lambda shp: pl.BlockSpec(shp, lambda i: (0,) * len(shp))
    return pl.pallas_call(
        _k1_body,
        grid=grid,
        in_specs=[
            pl.BlockSpec((R, 128), lambda i: (i, 0)),
            full2((128, H)), full2((H,)), full2((128, H)), full2((H,)),
            full2((2, H)), full2((H,)), full2((5000, 128)),
        ],
        out_specs=[
            pl.BlockSpec((R, H), lambda i: (i, 0)),
            pl.BlockSpec((R, H), lambda i: (i, 0)),
            pl.BlockSpec((R, HH), lambda i: (i, 0)),
            pl.BlockSpec((R, HH), lambda i: (i, 0)),
            pl.BlockSpec((R,), lambda i: (i,)),
            pl.BlockSpec((2, 128), lambda i: (0, 0)),
        ],
        out_shape=[
            jax.ShapeDtypeStruct((NP, H), jnp.float32),
            jax.ShapeDtypeStruct((NP, H), jnp.float32),
            jax.ShapeDtypeStruct((NP, HH), jnp.float32),
            jax.ShapeDtypeStruct((NP, HH), jnp.float32),
            jax.ShapeDtypeStruct((NP,), jnp.float32),
            jax.ShapeDtypeStruct((2, 128), jnp.float32),
        ],
        scratch_shapes=[pltpu.SMEM((2,), jnp.float32)],
    )(x_pad, W1l, b1l, W1r, b1r, We1, att1, ea_rs)


def _k4_body(accA_ref, accB_ref, denp_ref, selfw_ref,
             xlA_ref, xlB_ref, bias_ref,
             Wl_ref, bl_ref, Wr_ref, br_ref, We_ref, att_ref, mea_ref,
             axl_ref, axr_ref, xlA2_ref, xlB2_ref, selfw2_ref):
    selfw = selfw_ref[...]
    den = denp_ref[0, :] + denp_ref[1, :] + selfw
    inv = 1.0 / den
    hA = jnp.maximum(
        (accA_ref[...] + selfw[:, None] * xlA_ref[...]) * inv[:, None]
        + bias_ref[:HH][None, :], 0.0)
    hB = jnp.maximum(
        (accB_ref[...] + selfw[:, None] * xlB_ref[...]) * inv[:, None]
        + bias_ref[HH:][None, :], 0.0)
    Wl = Wl_ref[...]
    Wr = Wr_ref[...]
    xl = (jnp.dot(hA, Wl[:HH, :], preferred_element_type=jnp.float32)
          + jnp.dot(hB, Wl[HH:, :], preferred_element_type=jnp.float32) + bl_ref[...])
    xr = (jnp.dot(hA, Wr[:HH, :], preferred_element_type=jnp.float32)
          + jnp.dot(hB, Wr[HH:, :], preferred_element_type=jnp.float32) + br_ref[...])
    att = att_ref[...]
    axl = att[None, :] * xl
    axr = att[None, :] * xr
    aWe0 = att * We_ref[0, :]
    aWe1 = att * We_ref[1, :]
    s = jnp.sign(att)
    cself = mea_ref[0, 0] * aWe0 + mea_ref[1, 0] * aWe1
    u = axl + axr + cself[None, :]
    logit = 0.6 * jnp.sum(u, axis=1) + 0.4 * jnp.sum(s[None, :] * jnp.abs(u), axis=1)
    axl_ref[...] = axl
    axr_ref[...] = axr
    xlA2_ref[...] = xl[:, :HH]
    xlB2_ref[...] = xl[:, HH:]
    selfw2_ref[...] = jnp.exp(logit)


def _k4(accA, accB, denp, selfw, xlA, xlB, bias,
        Wl, bl, Wr, br, We, att, mea):
    grid = (NP // R,)
    full2 = lambda shp: pl.BlockSpec(shp, lambda i: (0,) * len(shp))
    return pl.pallas_call(
        _k4_body,
        grid=grid,
        in_specs=[
            pl.BlockSpec((R, HH), lambda i: (i, 0)),
            pl.BlockSpec((R, HH), lambda i: (i, 0)),
            pl.BlockSpec((2, R), lambda i: (0, i)),
            pl.BlockSpec((R,), lambda i: (i,)),
            pl.BlockSpec((R, HH), lambda i: (i, 0)),
            pl.BlockSpec((R, HH), lambda i: (i, 0)),
            full2((H,)),
            full2((H, H)), full2((H,)), full2((H, H)), full2((H,)),
            full2((2, H)), full2((H,)), full2((2, 128)),
        ],
        out_specs=[
            pl.BlockSpec((R, H), lambda i: (i, 0)),
            pl.BlockSpec((R, H), lambda i: (i, 0)),
            pl.BlockSpec((R, HH), lambda i: (i, 0)),
            pl.BlockSpec((R, HH), lambda i: (i, 0)),
            pl.BlockSpec((R,), lambda i: (i,)),
        ],
        out_shape=[
            jax.ShapeDtypeStruct((NP, H), jnp.float32),
            jax.ShapeDtypeStruct((NP, H), jnp.float32),
            jax.ShapeDtypeStruct((NP, HH), jnp.float32),
            jax.ShapeDtypeStruct((NP, HH), jnp.float32),
            jax.ShapeDtypeStruct((NP,), jnp.float32),
        ],
    )(accA, accB, denp, selfw, xlA, xlB, bias,
      Wl, bl, Wr, br, We, att, mea)


def _pass1_body(axl_h, axr_h, att_h, We_h, src_h, dst_h, ea0_h, ea1_h,
                w_h, denp_h,
                attv, wev, aWe0, aWe1, sv, srcL, dstL, wloc,
                hbufA, hbufB, rowvA, rowvB,
                bufLA, bufRA, bufLB, bufRB, ea0A, ea1A, ea0B, ea1B,
                shden, semA, semB, semHA, semHB):
    c = lax.axis_index("c")
    s = lax.axis_index("s")
    wid = s * 2 + c
    base0 = wid * EW1
    pltpu.sync_copy(att_h, attv)
    pltpu.sync_copy(We_h, wev)
    pltpu.sync_copy(src_h.at[pl.ds(base0, EW1)], srcL)
    pltpu.sync_copy(dst_h.at[pl.ds(base0, EW1)], dstL)
    for j in range(16):
        sl = pl.ds(j * 16, 16)
        a = attv[sl]
        aWe0[sl] = a * wev[0, sl]
        aWe1[sl] = a * wev[1, sl]
        sv[sl] = jnp.sign(a)

    lane_iota = lax.iota(jnp.int32, 16)
    zero16 = jnp.zeros((16,), jnp.float32)
    rot_idx = [(lane_iota + sh) & 15 for sh in (8, 4, 2, 1)]

    def rsum(v):
        for idx in rot_idx:
            v = v + v[idx]
        return v

    izero = jnp.zeros((16,), jnp.int32)

    def zrow(r, carry):
        for j in range(8):
            hbufA[r, pl.ds(j * 16, 16)] = zero16
            hbufB[r, pl.ds(j * 16, 16)] = zero16
        return carry

    lax.fori_loop(0, C1, zrow, 0, unroll=False)
    for j in range(C1 // 16):
        sl = pl.ds(j * 16, 16)
        rowvA[sl] = izero
        rowvB[sl] = izero
    pltpu.sync_copy(hbufA.at[pl.ds(0, DR // 16)],
                    shden.at[pl.ds(s * (DR // 16), DR // 16)])
    plsc.subcore_barrier()

    bufs = {0: (bufLA, bufRA, ea0A, ea1A, semA, hbufA, rowvA, semHA),
            1: (bufLB, bufRB, ea0B, ea1B, semB, hbufB, rowvB, semHB)}

    def fire_hist(par):
        bL, bR, e0b, e1b, sem, hb, rv, semH = bufs[par]
        pltpu.async_copy(hb, shden.at[rv], semH, add=True)

    def drain_hist(par):
        bL, bR, e0b, e1b, sem, hb, rv, semH = bufs[par]
        pltpu.make_async_copy(
            axl_h.at[pl.ds(0, C1), pl.ds(0, 128)], hb, semH).wait()

    def issue(cidx, par):
        bL, bR, e0b, e1b, sem = bufs[par][:5]
        pltpu.async_copy(axl_h.at[srcL.at[pl.ds(cidx * C1, C1)]], bL, sem)
        pltpu.async_copy(axr_h.at[dstL.at[pl.ds(cidx * C1, C1)]], bR, sem)
        pltpu.async_copy(ea0_h.at[pl.ds(base0 + cidx * C1, C1)], e0b, sem)
        pltpu.async_copy(ea1_h.at[pl.ds(base0 + cidx * C1, C1)], e1b, sem)

    def drain(par):
        bL, bR, e0b, e1b, sem = bufs[par][:5]
        pltpu.make_async_copy(axl_h.at[pl.ds(0, C1)], bL, sem).wait()
        pltpu.make_async_copy(axl_h.at[pl.ds(0, C1)], bR, sem).wait()
        pltpu.make_async_copy(ea0_h.at[pl.ds(0, C1)], e0b, sem).wait()
        pltpu.make_async_copy(ea0_h.at[pl.ds(0, C1)], e1b, sem).wait()

    def compute(cidx, par):
        bL, bR, e0b, e1b, sem, hbuf, rowv, semH = bufs[par]
        drain_hist(par)
        aWe0r = [aWe0[pl.ds(16 * j, 16)] for j in range(16)]
        aWe1r = [aWe1[pl.ds(16 * j, 16)] for j in range(16)]
        svr = [sv[pl.ds(16 * j, 16)] for j in range(16)]

        def group(g, carry):
            gb16 = g * 16
            e0g = e0b[pl.ds(gb16, 16)]
            e1g = e1b[pl.ds(gb16, 16)]
            dg = dstL[pl.ds(cidx * C1 + gb16, 16)]
            colg = jnp.bitwise_and(dg, 127)

            def lanefn(i, wcar):
                spl = jnp.full((16,), i, jnp.int32)
                e0v = e0g[spl]
                e1v = e1g[spl]
                el = gb16 + i
                lin = zero16
                ab = zero16
                for j in range(16):
                    sl = pl.ds(j * 16, 16)
                    u = bufs[par][0][el, sl] + bufs[par][1][el, sl] \
                        + e0v * aWe0r[j] + e1v * aWe1r[j]
                    lin = lin + u
                    ab = ab + svr[j] * jnp.abs(u)
                lv = rsum(0.6 * lin + 0.4 * ab)
                colv = colg[spl]
                expv = jnp.exp(lv)
                for j in range(8):
                    hbuf[el, pl.ds(j * 16, 16)] = jnp.where(
                        lane_iota + (j * 16) == colv, expv, 0.0)
                return jnp.where(lane_iota == i, lv, wcar)

            wvec = lax.fori_loop(0, 16, lanefn, zero16, unroll=False)
            wloc[pl.ds(cidx * C1 + gb16, 16)] = jnp.exp(wvec)
            return carry

        lax.fori_loop(0, C1 // 16, group, 0, unroll=False)
        for j in range(C1 // 16):
            sl = pl.ds(j * 16, 16)
            rowv[sl] = jnp.right_shift(dstL[pl.ds(cidx * C1 + j * 16, 16)], 7)
        fire_hist(par)

    fire_hist(0)
    fire_hist(1)
    issue(0, 0)

    def pair(p, carry):
        issue(2 * p + 1, 1)
        drain(0)
        compute(2 * p, 0)
        issue(2 * p + 2, 0)
        drain(1)
        compute(2 * p + 1, 1)
        return carry

    lax.fori_loop(0, NCH1 // 2 - 1, pair, 0, unroll=False)
    issue(NCH1 - 1, 1)
    drain(0)
    compute(NCH1 - 2, 0)
    drain(1)
    compute(NCH1 - 1, 1)
    drain_hist(0)
    drain_hist(1)

    pltpu.sync_copy(wloc, w_h.at[pl.ds(base0, EW1)])
    plsc.subcore_barrier()

    @pl.when(s == 0)
    def _():
        pltpu.sync_copy(shden, denp_h.at[pl.ds(c * DR, DR)])


def _pass1(axl, axr, att, We, src, dst, ea0, ea1):
    mesh = plsc.VectorSubcoreMesh(core_axis_name="c", subcore_axis_name="s")
    f = functools.partial(
        pl.kernel,
        out_type=[
            jax.ShapeDtypeStruct((EP,), jnp.float32),
            jax.ShapeDtypeStruct((2 * DR, 128), jnp.float32),
        ],
        mesh=mesh,
        scratch_types=[
            pltpu.VMEM((H,), jnp.float32),
            pltpu.VMEM((2, H), jnp.float32),
            pltpu.VMEM((H,), jnp.float32),
            pltpu.VMEM((H,), jnp.float32),
            pltpu.VMEM((H,), jnp.float32),
            pltpu.VMEM((EW1,), jnp.int32),
            pltpu.VMEM((EW1,), jnp.int32),
            pltpu.VMEM((EW1,), jnp.float32),
            pltpu.VMEM((C1, 128), jnp.float32),
            pltpu.VMEM((C1, 128), jnp.float32),
            pltpu.VMEM((C1,), jnp.int32),
            pltpu.VMEM((C1,), jnp.int32),
            pltpu.VMEM((C1, H), jnp.float32),
            pltpu.VMEM((C1, H), jnp.float32),
            pltpu.VMEM((C1, H), jnp.float32),
            pltpu.VMEM((C1, H), jnp.float32),
            pltpu.VMEM((C1,), jnp.float32),
            pltpu.VMEM((C1,), jnp.float32),
            pltpu.VMEM((C1,), jnp.float32),
            pltpu.VMEM((C1,), jnp.float32),
            pltpu.VMEM_SHARED((DR, 128), jnp.float32),
            pltpu.SemaphoreType.DMA,
            pltpu.SemaphoreType.DMA,
            pltpu.SemaphoreType.DMA,
            pltpu.SemaphoreType.DMA,
        ],
    )(_pass1_body)
    return f(axl, axr, att, We, src, dst, ea0, ea1)


def _kden_body(denp_ref, den_ref):
    den_ref[...] = denp_ref[0, :] + denp_ref[1, :]


def _kden(denp):
    return pl.pallas_call(
        _kden_body,
        grid=(NP // R,),
        in_specs=[pl.BlockSpec((2, R), lambda i: (0, i))],
        out_specs=pl.BlockSpec((R,), lambda i: (i,)),
        out_shape=jax.ShapeDtypeStruct((NP,), jnp.float32),
    )(denp)


def _pass2_body(xlF_h, srcB_h, dst_h, w_h,
                acc_h,
                shacc, srcLoc, dstvA, dstvB, dscA, dscB, wvA, wvB,
                gbufA, gbufB, sbufA, sbufB,
                semA, semB, semSA, semSB):
    c = lax.axis_index("c")
    s = lax.axis_index("s")
    zero16 = jnp.zeros((16,), jnp.float32)
    izero = jnp.zeros((16,), jnp.int32)

    def zrow(r, carry):
        for j in range(8):
            sbufA[r, pl.ds(j * 16, 16)] = zero16
            sbufB[r, pl.ds(j * 16, 16)] = zero16
        return carry

    lax.fori_loop(0, C2, zrow, 0, unroll=False)
    for t in range(10):
        pltpu.sync_copy(sbufA, shacc.at[pl.ds(s * 640 + t * 64, 64)])
    for j in range(C2 // 16):
        sl = pl.ds(j * 16, 16)
        dstvA[sl] = izero
        dstvB[sl] = izero
        dscA[sl] = izero
        dscB[sl] = izero
    plsc.subcore_barrier()

    base0 = s * EW2
    bufs = {0: (gbufA, dstvA, dscA, wvA, sbufA, semA, semSA),
            1: (gbufB, dstvB, dscB, wvB, sbufB, semB, semSB)}

    def drain(par):
        gb, dv, dc, wv, sb, sem, semS = bufs[par]
        pltpu.make_async_copy(xlF_h.at[pl.ds(0, C2)], gb, sem).wait()
        pltpu.make_async_copy(dst_h.at[pl.ds(0, C2)], dv, sem).wait()
        pltpu.make_async_copy(w_h.at[pl.ds(0, C2)], wv, sem).wait()

    def drain_scatter(par):
        gb, dv, dc, wv, sb, sem, semS = bufs[par]
        pltpu.make_async_copy(xlF_h.at[pl.ds(0, C2)], sb, semS).wait()

    def fire_scatter(par):
        gb, dv, dc, wv, sb, sem, semS = bufs[par]
        pltpu.async_copy(sb, shacc.at[dc], semS, add=True)

    def compute(cidx_unused, par):
        gb, dv, dc, wv, sb, sem, semS = bufs[par]
        for j in range(C2 // 16):
            sl = pl.ds(j * 16, 16)
            dc[sl] = dv[sl]

        def group(g, carry2):
            wg = wv[pl.ds(g * 16, 16)]
            for i in range(16):
                e = g * 16 + i
                we = wg[i]
                for j in range(8):
                    sl = pl.ds(j * 16, 16)
                    sb[e, sl] = we * gb[e, sl]
            return carry2

        lax.fori_loop(0, C2 // 16, group, 0, unroll=False)

    fire_scatter(0)
    fire_scatter(1)

    def phase(ph, carry):
        pbase = base0 + ph * EW2H
        pltpu.sync_copy(srcB_h.at[pl.ds(c * EP + pbase, EW2H)], srcLoc)

        def issue(cidx, par):
            gb, dv, dc, wv, sb, sem, semS = bufs[par]
            pltpu.async_copy(
                xlF_h.at[srcLoc.at[pl.ds(cidx * C2, C2)]], gb, sem)
            pltpu.async_copy(dst_h.at[pl.ds(pbase + cidx * C2, C2)], dv, sem)
            pltpu.async_copy(w_h.at[pl.ds(pbase + cidx * C2, C2)], wv, sem)

        issue(0, 0)

        def pair(p, carry2):
            issue(2 * p + 1, 1)
            drain(0)
            drain_scatter(0)
            compute(2 * p, 0)
            fire_scatter(0)
            issue(2 * p + 2, 0)
            drain(1)
            drain_scatter(1)
            compute(2 * p + 1, 1)
            fire_scatter(1)
            return carry2

        lax.fori_loop(0, NCHP // 2 - 1, pair, 0, unroll=False)
        issue(NCHP - 1, 1)
        drain(0)
        drain_scatter(0)
        compute(NCHP - 2, 0)
        fire_scatter(0)
        drain(1)
        drain_scatter(1)
        compute(NCHP - 1, 1)
        fire_scatter(1)
        return carry

    lax.fori_loop(0, 2, phase, 0, unroll=False)
    drain_scatter(0)
    drain_scatter(1)
    plsc.subcore_barrier()
    pltpu.sync_copy(shacc.at[pl.ds(s * 640, 640)],
                    acc_h.at[pl.ds(c * NP + s * 640, 640)])


def _pass2(xl_flat, srcB, dst, w):
    mesh = plsc.VectorSubcoreMesh(core_axis_name="c", subcore_axis_name="s")
    f = functools.partial(
        pl.kernel,
        out_type=jax.ShapeDtypeStruct((2 * NP, HH), jnp.float32),
        mesh=mesh,
        scratch_types=[
            pltpu.VMEM_SHARED((NP, HH), jnp.float32),
            pltpu.VMEM((EW2H,), jnp.int32),
            pltpu.VMEM((C2,), jnp.int32),
            pltpu.VMEM((C2,), jnp.int32),
            pltpu.VMEM((C2,), jnp.int32),
            pltpu.VMEM((C2,), jnp.int32),
            pltpu.VMEM((C2,), jnp.float32),
            pltpu.VMEM((C2,), jnp.float32),
            pltpu.VMEM((C2, HH), jnp.float32),
            pltpu.VMEM((C2, HH), jnp.float32),
            pltpu.VMEM((C2, HH), jnp.float32),
            pltpu.VMEM((C2, HH), jnp.float32),
            pltpu.SemaphoreType.DMA,
            pltpu.SemaphoreType.DMA,
            pltpu.SemaphoreType.DMA,
            pltpu.SemaphoreType.DMA,
        ],
    )(_pass2_body)
    return f(xl_flat, srcB, dst, w)


def _pool_body(accA_h, accB_h, den_h, selfw_h, xlA_h, xlB_h,
               bias_h, batch_h,
               psum_h, pmax_h, pcnt_h,
               biasv, batv, swv, d0v, bA, bB, bxA, bxB,
               gsum, gmax, gcnt):
    c = lax.axis_index("c")
    s = lax.axis_index("s")
    wid = s * 2 + c
    lane_iota = lax.iota(jnp.int32, 16)
    lane0 = jnp.where(lane_iota == 0, 1.0, 0.0)
    pltpu.sync_copy(bias_h, biasv)

    def zrow(g, carry):
        for j in range(16):
            sl = pl.ds(j * 16, 16)
            gsum[g, sl] = jnp.zeros((16,), jnp.float32)
            gmax[g, sl] = jnp.full((16,), NEG, jnp.float32)
        gcnt[g, pl.ds(0, 16)] = jnp.zeros((16,), jnp.float32)
        return carry

    lax.fori_loop(0, G, zrow, 0, unroll=False)

    def chunkfn(t, carry):
        base = wid * RW + t * PC
        pltpu.sync_copy(accA_h.at[pl.ds(base, PC)], bA)
        pltpu.sync_copy(accB_h.at[pl.ds(base, PC)], bB)
        pltpu.sync_copy(den_h.at[pl.ds(base, PC)], d0v)
        pltpu.sync_copy(selfw_h.at[pl.ds(base, PC)], swv)
        pltpu.sync_copy(xlA_h.at[pl.ds(base, PC)], bxA)
        pltpu.sync_copy(xlB_h.at[pl.ds(base, PC)], bxB)
        pltpu.sync_copy(batch_h.at[pl.ds(base, PC)], batv)

        def rowgroup(gg, carry2):
            batg = batv[pl.ds(gg * 16, 16)]
            swg = swv[pl.ds(gg * 16, 16)]
            d0g = d0v[pl.ds(gg * 16, 16)]
            invg = 1.0 / (d0g + swg)
            for i in range(16):
                r = gg * 16 + i
                b = batg[i]
                sw = swg[i]
                inv = invg[i]

                @pl.when(b < G)
                def _():
                    for j in range(8):
                        sl = pl.ds(j * 16, 16)
                        slB = pl.ds(HH + j * 16, 16)
                        hA = jnp.maximum(
                            (bA[r, sl] + sw * bxA[r, sl]) * inv + biasv[sl], 0.0)
                        hB = jnp.maximum(
                            (bB[r, sl] + sw * bxB[r, sl]) * inv + biasv[slB], 0.0)
                        gsum[b, sl] = gsum[b, sl] + hA
                        gsum[b, slB] = gsum[b, slB] + hB
                        gmax[b, sl] = jnp.maximum(gmax[b, sl], hA)
                        gmax[b, slB] = jnp.maximum(gmax[b, slB], hB)
                    gcnt[b, pl.ds(0, 16)] = gcnt[b, pl.ds(0, 16)] + lane0

            return carry2

        lax.fori_loop(0, PC // 16, rowgroup, 0, unroll=False)
        return carry

    lax.fori_loop(0, RW // PC, chunkfn, 0, unroll=False)
    pltpu.sync_copy(gsum, psum_h.at[pl.ds(wid * G, G)])
    pltpu.sync_copy(gmax, pmax_h.at[pl.ds(wid * G, G)])
    pltpu.sync_copy(gcnt, pcnt_h.at[pl.ds(wid * G, G)])


def _pool(accA, accB, den, selfw, xlA, xlB, bias, batch_pad):
    mesh = plsc.VectorSubcoreMesh(core_axis_name="c", subcore_axis_name="s")
    f = functools.partial(
        pl.kernel,
        out_type=[
            jax.ShapeDtypeStruct((32 * G, H), jnp.float32),
            jax.ShapeDtypeStruct((32 * G, H), jnp.float32),
            jax.ShapeDtypeStruct((32 * G, 16), jnp.float32),
        ],
        mesh=mesh,
        scratch_types=[
            pltpu.VMEM((H,), jnp.float32),
            pltpu.VMEM((PC,), jnp.int32),
            pltpu.VMEM((PC,), jnp.float32),
            pltpu.VMEM((PC,), jnp.float32),
            pltpu.VMEM((PC, HH), jnp.float32),
            pltpu.VMEM((PC, HH), jnp.float32),
            pltpu.VMEM((PC, HH), jnp.float32),
            pltpu.VMEM((PC, HH), jnp.float32),
            pltpu.VMEM((G, H), jnp.float32),
            pltpu.VMEM((G, H), jnp.float32),
            pltpu.VMEM((G, 16), jnp.float32),
        ],
    )(_pool_body)
    return f(accA, accB, den, selfw, xlA, xlB, bias, batch_pad)


def _k9_body(psum_ref, pmax_ref, pcnt_ref, fcW_ref, fcb_ref, out_ref):
    gsum = jnp.sum(psum_ref[...], axis=0)
    gmax = jnp.max(pmax_ref[...], axis=0)
    counts = jnp.sum(pcnt_ref[...], axis=(0, 2))
    gmean = gsum / jnp.maximum(counts, 1.0)[:, None]
    gmaxz = jnp.where(counts[:, None] > 0, gmax, 0.0)
    fcW = fcW_ref[...]
    out = (jnp.dot(gmaxz, fcW[:H, :], preferred_element_type=jnp.float32)
           + jnp.dot(gmean, fcW[H:, :], preferred_element_type=jnp.float32)
           + fcb_ref[0])
    out_ref[...] = out


def _k9(psum, pmax, pcnt, fcW, fcb):
    return pl.pallas_call(
        _k9_body,
        out_shape=jax.ShapeDtypeStruct((G, 1), jnp.float32),
    )(psum, pmax, pcnt, fcW, fcb)


def kernel(x, edge_attr, W1l, b1l, W1r, b1r, We1, att1, bias1,
           W2l, b2l, W2r, b2r, We2, att2, bias2, fcW, fcb,
           edge_index, batch):
    x_pad = jnp.pad(x, ((0, NP - N), (0, 0)))
    src = jnp.concatenate([edge_index[0], jnp.full((EP - E,), N, jnp.int32)])
    dst = jnp.concatenate([edge_index[1], jnp.full((EP - E,), N, jnp.int32)])
    ea0 = jnp.concatenate([edge_attr[:, 0], jnp.zeros((EP - E,), jnp.float32)])
    ea1 = jnp.concatenate([edge_attr[:, 1], jnp.zeros((EP - E,), jnp.float32)])
    ea_rs = jnp.reshape(edge_attr, (5000, 128))
    batch_pad = jnp.concatenate([batch, jnp.full((NP - N,), G, jnp.int32)])

    axl1, axr1, xlA1, xlB1, selfw1, mea = _k1(
        x_pad, W1l, b1l, W1r, b1r, We1, att1, ea_rs)
    w1, denp1 = _pass1(axl1, axr1, att1, We1, src, dst, ea0, ea1)
    denp1 = jnp.reshape(denp1, (2, NP))
    srcB = jnp.concatenate([src, src + NP])
    xl1_flat = jnp.concatenate([xlA1, xlB1], axis=0)
    acc1 = _pass2(xl1_flat, srcB, dst, w1)

    axl2, axr2, xlA2, xlB2, selfw2 = _k4(
        acc1[:NP], acc1[NP:], denp1, selfw1, xlA1, xlB1, bias1,
        W2l, b2l, W2r, b2r, We2, att2, mea)
    w2, denp2 = _pass1(axl2, axr2, att2, We2, src, dst, ea0, ea1)
    denp2 = jnp.reshape(denp2, (2, NP))
    xl2_flat = jnp.concatenate([xlA2, xlB2], axis=0)
    acc2 = _pass2(xl2_flat, srcB, dst, w2)
    den2 = _kden(denp2)

    psum, pmax, pcnt = _pool(
        acc2[:NP], acc2[NP:], den2, selfw2, xlA2, xlB2,
        bias2, batch_pad)
    psum = jnp.reshape(psum, (32, G, H))
    pmax = jnp.reshape(pmax, (32, G, H))
    pcnt = jnp.reshape(pcnt, (32, G, 16))
    return _k9(psum, pmax, pcnt, fcW, fcb)


# speedup vs baseline: 1.0906x; 1.0204x over previous
import functools

import jax
import jax.numpy as jnp
from jax import lax
from jax.experimental import pallas as pl
from jax.experimental.pallas import tpu as pltpu
from jax.experimental.pallas import tpu_sc as plsc

N = 10000
NP = 10240
E = 320000
EP = 323584
HH = 128
H = 256
G = 64
R = 512
C1 = 64
EW1 = EP // 32
NCH1 = EW1 // C1
C2 = 64
EW2 = EP // 16
EW2H = EW2 // 2
NCHP = EW2H // C2
RW = NP // 32
PC = 80
DR = NP // 128
NEG = -3.0e38


def _k1_body(x_ref, Wl_ref, bl_ref, Wr_ref, br_ref, We_ref, att_ref, ea_ref,
             axl_ref, axr_ref, xlA_ref, xlB_ref, selfw_ref, mea_ref, mea_smem):
    i = pl.program_id(0)

    @pl.when(i == 0)
    def _():
        sv = jnp.sum(ea_ref[...], axis=0)
        par = lax.broadcasted_iota(jnp.int32, (128,), 0) % 2
        mea_smem[0] = jnp.sum(jnp.where(par == 0, sv, 0.0)) / E
        mea_smem[1] = jnp.sum(jnp.where(par == 1, sv, 0.0)) / E

    h = jnp.maximum(x_ref[...], 0.0)
    att = att_ref[...]
    xl = jnp.dot(h, Wl_ref[...], preferred_element_type=jnp.float32) + bl_ref[...]
    xr = jnp.dot(h, Wr_ref[...], preferred_element_type=jnp.float32) + br_ref[...]
    axl = att[None, :] * xl
    axr = att[None, :] * xr
    aWe0 = att * We_ref[0, :]
    aWe1 = att * We_ref[1, :]
    s = jnp.sign(att)
    cself = mea_smem[0] * aWe0 + mea_smem[1] * aWe1
    u = axl + axr + cself[None, :]
    logit = 0.6 * jnp.sum(u, axis=1) + 0.4 * jnp.sum(s[None, :] * jnp.abs(u), axis=1)
    axl_ref[...] = axl
    axr_ref[...] = axr
    xlA_ref[...] = xl[:, :HH]
    xlB_ref[...] = xl[:, HH:]
    selfw_ref[...] = jnp.exp(logit)
    mea_ref[...] = jnp.concatenate(
        [jnp.full((1, 128), mea_smem[0], jnp.float32),
         jnp.full((1, 128), mea_smem[1], jnp.float32)], axis=0)


def _k1(x_pad, W1l, b1l, W1r, b1r, We1, att1, ea_rs):
    grid = (NP // R,)
    full2 = lambda shp: pl.BlockSpec(shp, lambda i: (0,) * len(shp))
    return pl.pallas_call(
        _k1_body,
        grid=grid,
        in_specs=[
            pl.BlockSpec((R, 128), lambda i: (i, 0)),
            full2((128, H)), full2((H,)), full2((128, H)), full2((H,)),
            full2((2, H)), full2((H,)), full2((5000, 128)),
        ],
        out_specs=[
            pl.BlockSpec((R, H), lambda i: (i, 0)),
            pl.BlockSpec((R, H), lambda i: (i, 0)),
            pl.BlockSpec((R, HH), lambda i: (i, 0)),
            pl.BlockSpec((R, HH), lambda i: (i, 0)),
            pl.BlockSpec((R,), lambda i: (i,)),
            pl.BlockSpec((2, 128), lambda i: (0, 0)),
        ],
        out_shape=[
            jax.ShapeDtypeStruct((NP, H), jnp.float32),
            jax.ShapeDtypeStruct((NP, H), jnp.float32),
            jax.ShapeDtypeStruct((NP, HH), jnp.float32),
            jax.ShapeDtypeStruct((NP, HH), jnp.float32),
            jax.ShapeDtypeStruct((NP,), jnp.float32),
            jax.ShapeDtypeStruct((2, 128), jnp.float32),
        ],
        scratch_shapes=[pltpu.SMEM((2,), jnp.float32)],
    )(x_pad, W1l, b1l, W1r, b1r, We1, att1, ea_rs)


def _k4_body(accA_ref, accB_ref, denp_ref, selfw_ref,
             xlA_ref, xlB_ref, bias_ref,
             Wl_ref, bl_ref, Wr_ref, br_ref, We_ref, att_ref, mea_ref,
             axl_ref, axr_ref, xlA2_ref, xlB2_ref, selfw2_ref):
    selfw = selfw_ref[...]
    den = denp_ref[0, :] + denp_ref[1, :] + selfw
    inv = 1.0 / den
    hA = jnp.maximum(
        (accA_ref[...] + selfw[:, None] * xlA_ref[...]) * inv[:, None]
        + bias_ref[:HH][None, :], 0.0)
    hB = jnp.maximum(
        (accB_ref[...] + selfw[:, None] * xlB_ref[...]) * inv[:, None]
        + bias_ref[HH:][None, :], 0.0)
    Wl = Wl_ref[...]
    Wr = Wr_ref[...]
    xl = (jnp.dot(hA, Wl[:HH, :], preferred_element_type=jnp.float32)
          + jnp.dot(hB, Wl[HH:, :], preferred_element_type=jnp.float32) + bl_ref[...])
    xr = (jnp.dot(hA, Wr[:HH, :], preferred_element_type=jnp.float32)
          + jnp.dot(hB, Wr[HH:, :], preferred_element_type=jnp.float32) + br_ref[...])
    att = att_ref[...]
    axl = att[None, :] * xl
    axr = att[None, :] * xr
    aWe0 = att * We_ref[0, :]
    aWe1 = att * We_ref[1, :]
    s = jnp.sign(att)
    cself = mea_ref[0, 0] * aWe0 + mea_ref[1, 0] * aWe1
    u = axl + axr + cself[None, :]
    logit = 0.6 * jnp.sum(u, axis=1) + 0.4 * jnp.sum(s[None, :] * jnp.abs(u), axis=1)
    axl_ref[...] = axl
    axr_ref[...] = axr
    xlA2_ref[...] = xl[:, :HH]
    xlB2_ref[...] = xl[:, HH:]
    selfw2_ref[...] = jnp.exp(logit)


def _k4(accA, accB, denp, selfw, xlA, xlB, bias,
        Wl, bl, Wr, br, We, att, mea):
    grid = (NP // R,)
    full2 = lambda shp: pl.BlockSpec(shp, lambda i: (0,) * len(shp))
    return pl.pallas_call(
        _k4_body,
        grid=grid,
        in_specs=[
            pl.BlockSpec((R, HH), lambda i: (i, 0)),
            pl.BlockSpec((R, HH), lambda i: (i, 0)),
            pl.BlockSpec((2, R), lambda i: (0, i)),
            pl.BlockSpec((R,), lambda i: (i,)),
            pl.BlockSpec((R, HH), lambda i: (i, 0)),
            pl.BlockSpec((R, HH), lambda i: (i, 0)),
            full2((H,)),
            full2((H, H)), full2((H,)), full2((H, H)), full2((H,)),
            full2((2, H)), full2((H,)), full2((2, 128)),
        ],
        out_specs=[
            pl.BlockSpec((R, H), lambda i: (i, 0)),
            pl.BlockSpec((R, H), lambda i: (i, 0)),
            pl.BlockSpec((R, HH), lambda i: (i, 0)),
            pl.BlockSpec((R, HH), lambda i: (i, 0)),
            pl.BlockSpec((R,), lambda i: (i,)),
        ],
        out_shape=[
            jax.ShapeDtypeStruct((NP, H), jnp.float32),
            jax.ShapeDtypeStruct((NP, H), jnp.float32),
            jax.ShapeDtypeStruct((NP, HH), jnp.float32),
            jax.ShapeDtypeStruct((NP, HH), jnp.float32),
            jax.ShapeDtypeStruct((NP,), jnp.float32),
        ],
    )(accA, accB, denp, selfw, xlA, xlB, bias,
      Wl, bl, Wr, br, We, att, mea)


def _pass1_body(axl_h, axr_h, att_h, We_h, src_h, dst_h, ea0_h, ea1_h,
                w_h, denp_h,
                attv, wev, aWe0, aWe1, sv, srcL, dstL, wloc,
                hbufA, hbufB, rowvA, rowvB,
                bufLA, bufRA, bufLB, bufRB, ea0A, ea1A, ea0B, ea1B,
                shden, semA, semB, semHA, semHB):
    c = lax.axis_index("c")
    s = lax.axis_index("s")
    wid = s * 2 + c
    base0 = wid * EW1
    pltpu.sync_copy(att_h, attv)
    pltpu.sync_copy(We_h, wev)
    pltpu.sync_copy(src_h.at[pl.ds(base0, EW1)], srcL)
    pltpu.sync_copy(dst_h.at[pl.ds(base0, EW1)], dstL)
    for j in range(16):
        sl = pl.ds(j * 16, 16)
        a = attv[sl]
        aWe0[sl] = a * wev[0, sl]
        aWe1[sl] = a * wev[1, sl]
        sv[sl] = jnp.sign(a)

    lane_iota = lax.iota(jnp.int32, 16)
    zero16 = jnp.zeros((16,), jnp.float32)
    rot_idx = [(lane_iota + sh) & 15 for sh in (8, 4, 2, 1)]

    def rsum(v):
        for idx in rot_idx:
            v = v + v[idx]
        return v

    izero = jnp.zeros((16,), jnp.int32)

    def zrow(r, carry):
        for j in range(8):
            hbufA[r, pl.ds(j * 16, 16)] = zero16
            hbufB[r, pl.ds(j * 16, 16)] = zero16
        return carry

    lax.fori_loop(0, C1, zrow, 0, unroll=False)
    for j in range(C1 // 16):
        sl = pl.ds(j * 16, 16)
        rowvA[sl] = izero
        rowvB[sl] = izero
    pltpu.sync_copy(hbufA.at[pl.ds(0, DR // 16)],
                    shden.at[pl.ds(s * (DR // 16), DR // 16)])
    plsc.subcore_barrier()

    bufs = {0: (bufLA, bufRA, ea0A, ea1A, semA, hbufA, rowvA, semHA),
            1: (bufLB, bufRB, ea0B, ea1B, semB, hbufB, rowvB, semHB)}

    def fire_hist(par):
        bL, bR, e0b, e1b, sem, hb, rv, semH = bufs[par]
        pltpu.async_copy(hb, shden.at[rv], semH, add=True)

    def drain_hist(par):
        bL, bR, e0b, e1b, sem, hb, rv, semH = bufs[par]
        pltpu.make_async_copy(
            axl_h.at[pl.ds(0, C1), pl.ds(0, 128)], hb, semH).wait()

    def issue(cidx, par):
        bL, bR, e0b, e1b, sem = bufs[par][:5]
        pltpu.async_copy(axl_h.at[srcL.at[pl.ds(cidx * C1, C1)]], bL, sem)
        pltpu.async_copy(axr_h.at[dstL.at[pl.ds(cidx * C1, C1)]], bR, sem)
        pltpu.async_copy(ea0_h.at[pl.ds(base0 + cidx * C1, C1)], e0b, sem)
        pltpu.async_copy(ea1_h.at[pl.ds(base0 + cidx * C1, C1)], e1b, sem)

    def drain(par):
        bL, bR, e0b, e1b, sem = bufs[par][:5]
        pltpu.make_async_copy(axl_h.at[pl.ds(0, C1)], bL, sem).wait()
        pltpu.make_async_copy(axl_h.at[pl.ds(0, C1)], bR, sem).wait()
        pltpu.make_async_copy(ea0_h.at[pl.ds(0, C1)], e0b, sem).wait()
        pltpu.make_async_copy(ea0_h.at[pl.ds(0, C1)], e1b, sem).wait()

    def compute(cidx, par):
        bL, bR, e0b, e1b, sem, hbuf, rowv, semH = bufs[par]
        drain_hist(par)
        aWe0r = [aWe0[pl.ds(16 * j, 16)] for j in range(16)]
        aWe1r = [aWe1[pl.ds(16 * j, 16)] for j in range(16)]
        svr = [sv[pl.ds(16 * j, 16)] for j in range(16)]

        def edge(el, wcar):
            gb16 = jnp.bitwise_and(el, -16)
            lane = jnp.bitwise_and(el, 15)
            spl = jnp.full((16,), lane, jnp.int32)
            e0v = e0b[pl.ds(gb16, 16)][spl]
            e1v = e1b[pl.ds(gb16, 16)][spl]
            lin = zero16
            ab = zero16
            for j in range(16):
                sl = pl.ds(j * 16, 16)
                u = bufs[par][0][el, sl] + bufs[par][1][el, sl] \
                    + e0v * aWe0r[j] + e1v * aWe1r[j]
                lin = lin + u
                ab = ab + svr[j] * jnp.abs(u)
            lv = rsum(0.6 * lin + 0.4 * ab)
            dvec = dstL[pl.ds(cidx * C1 + gb16, 16)][spl]
            colv = jnp.bitwise_and(dvec, 127)
            expv = jnp.exp(lv)
            for j in range(8):
                hbuf[el, pl.ds(j * 16, 16)] = jnp.where(
                    lane_iota + (j * 16) == colv, expv, 0.0)
            wcar = jnp.where(lane_iota == (el & 15), lv, wcar)

            @pl.when((el & 15) == 15)
            def _():
                wloc[pl.ds(cidx * C1 + el - 15, 16)] = jnp.exp(wcar)

            return wcar

        lax.fori_loop(0, C1, edge, zero16, unroll=False)
        for j in range(C1 // 16):
            sl = pl.ds(j * 16, 16)
            rowv[sl] = jnp.right_shift(dstL[pl.ds(cidx * C1 + j * 16, 16)], 7)
        fire_hist(par)

    fire_hist(0)
    fire_hist(1)
    issue(0, 0)

    def pair(p, carry):
        issue(2 * p + 1, 1)
        drain(0)
        compute(2 * p, 0)
        issue(2 * p + 2, 0)
        drain(1)
        compute(2 * p + 1, 1)
        return carry

    lax.fori_loop(0, NCH1 // 2 - 1, pair, 0, unroll=False)
    issue(NCH1 - 1, 1)
    drain(0)
    compute(NCH1 - 2, 0)
    drain(1)
    compute(NCH1 - 1, 1)
    drain_hist(0)
    drain_hist(1)

    pltpu.sync_copy(wloc, w_h.at[pl.ds(base0, EW1)])
    plsc.subcore_barrier()

    @pl.when(s == 0)
    def _():
        pltpu.sync_copy(shden, denp_h.at[pl.ds(c * DR, DR)])


def _pass1(axl, axr, att, We, src, dst, ea0, ea1):
    mesh = plsc.VectorSubcoreMesh(core_axis_name="c", subcore_axis_name="s")
    f = functools.partial(
        pl.kernel,
        out_type=[
            jax.ShapeDtypeStruct((EP,), jnp.float32),
            jax.ShapeDtypeStruct((2 * DR, 128), jnp.float32),
        ],
        mesh=mesh,
        scratch_types=[
            pltpu.VMEM((H,), jnp.float32),
            pltpu.VMEM((2, H), jnp.float32),
            pltpu.VMEM((H,), jnp.float32),
            pltpu.VMEM((H,), jnp.float32),
            pltpu.VMEM((H,), jnp.float32),
            pltpu.VMEM((EW1,), jnp.int32),
            pltpu.VMEM((EW1,), jnp.int32),
            pltpu.VMEM((EW1,), jnp.float32),
            pltpu.VMEM((C1, 128), jnp.float32),
            pltpu.VMEM((C1, 128), jnp.float32),
            pltpu.VMEM((C1,), jnp.int32),
            pltpu.VMEM((C1,), jnp.int32),
            pltpu.VMEM((C1, H), jnp.float32),
            pltpu.VMEM((C1, H), jnp.float32),
            pltpu.VMEM((C1, H), jnp.float32),
            pltpu.VMEM((C1, H), jnp.float32),
            pltpu.VMEM((C1,), jnp.float32),
            pltpu.VMEM((C1,), jnp.float32),
            pltpu.VMEM((C1,), jnp.float32),
            pltpu.VMEM((C1,), jnp.float32),
            pltpu.VMEM_SHARED((DR, 128), jnp.float32),
            pltpu.SemaphoreType.DMA,
            pltpu.SemaphoreType.DMA,
            pltpu.SemaphoreType.DMA,
            pltpu.SemaphoreType.DMA,
        ],
    )(_pass1_body)
    return f(axl, axr, att, We, src, dst, ea0, ea1)


def _kden_body(denp_ref, den_ref):
    den_ref[...] = denp_ref[0, :] + denp_ref[1, :]


def _kden(denp):
    return pl.pallas_call(
        _kden_body,
        grid=(NP // R,),
        in_specs=[pl.BlockSpec((2, R), lambda i: (0, i))],
        out_specs=pl.BlockSpec((R,), lambda i: (i,)),
        out_shape=jax.ShapeDtypeStruct((NP,), jnp.float32),
    )(denp)


def _pass2_body(xlF_h, srcB_h, dst_h, w_h,
                acc_h,
                shacc, srcLoc, dstvA, dstvB, dscA, dscB, wvA, wvB,
                gbufA, gbufB, sbufA, sbufB,
                semA, semB, semSA, semSB):
    c = lax.axis_index("c")
    s = lax.axis_index("s")
    zero16 = jnp.zeros((16,), jnp.float32)
    izero = jnp.zeros((16,), jnp.int32)

    def zrow(r, carry):
        for j in range(8):
            sbufA[r, pl.ds(j * 16, 16)] = zero16
            sbufB[r, pl.ds(j * 16, 16)] = zero16
        return carry

    lax.fori_loop(0, C2, zrow, 0, unroll=False)
    for t in range(10):
        pltpu.sync_copy(sbufA, shacc.at[pl.ds(s * 640 + t * 64, 64)])
    for j in range(C2 // 16):
        sl = pl.ds(j * 16, 16)
        dstvA[sl] = izero
        dstvB[sl] = izero
        dscA[sl] = izero
        dscB[sl] = izero
    plsc.subcore_barrier()

    base0 = s * EW2
    bufs = {0: (gbufA, dstvA, dscA, wvA, sbufA, semA, semSA),
            1: (gbufB, dstvB, dscB, wvB, sbufB, semB, semSB)}

    def drain(par):
        gb, dv, dc, wv, sb, sem, semS = bufs[par]
        pltpu.make_async_copy(xlF_h.at[pl.ds(0, C2)], gb, sem).wait()
        pltpu.make_async_copy(dst_h.at[pl.ds(0, C2)], dv, sem).wait()
        pltpu.make_async_copy(w_h.at[pl.ds(0, C2)], wv, sem).wait()

    def drain_scatter(par):
        gb, dv, dc, wv, sb, sem, semS = bufs[par]
        pltpu.make_async_copy(xlF_h.at[pl.ds(0, C2)], sb, semS).wait()

    def fire_scatter(par):
        gb, dv, dc, wv, sb, sem, semS = bufs[par]
        pltpu.async_copy(sb, shacc.at[dc], semS, add=True)

    def compute(cidx_unused, par):
        gb, dv, dc, wv, sb, sem, semS = bufs[par]
        for j in range(C2 // 16):
            sl = pl.ds(j * 16, 16)
            dc[sl] = dv[sl]

        def group(g, carry2):
            wg = wv[pl.ds(g * 16, 16)]
            for i in range(16):
                e = g * 16 + i
                we = wg[i]
                for j in range(8):
                    sl = pl.ds(j * 16, 16)
                    sb[e, sl] = we * gb[e, sl]
            return carry2

        lax.fori_loop(0, C2 // 16, group, 0, unroll=False)

    fire_scatter(0)
    fire_scatter(1)

    def phase(ph, carry):
        pbase = base0 + ph * EW2H
        pltpu.sync_copy(srcB_h.at[pl.ds(c * EP + pbase, EW2H)], srcLoc)

        def issue(cidx, par):
            gb, dv, dc, wv, sb, sem, semS = bufs[par]
            pltpu.async_copy(
                xlF_h.at[srcLoc.at[pl.ds(cidx * C2, C2)]], gb, sem)
            pltpu.async_copy(dst_h.at[pl.ds(pbase + cidx * C2, C2)], dv, sem)
            pltpu.async_copy(w_h.at[pl.ds(pbase + cidx * C2, C2)], wv, sem)

        issue(0, 0)

        def pair(p, carry2):
            issue(2 * p + 1, 1)
            drain(0)
            drain_scatter(0)
            compute(2 * p, 0)
            fire_scatter(0)
            issue(2 * p + 2, 0)
            drain(1)
            drain_scatter(1)
            compute(2 * p + 1, 1)
            fire_scatter(1)
            return carry2

        lax.fori_loop(0, NCHP // 2 - 1, pair, 0, unroll=False)
        issue(NCHP - 1, 1)
        drain(0)
        drain_scatter(0)
        compute(NCHP - 2, 0)
        fire_scatter(0)
        drain(1)
        drain_scatter(1)
        compute(NCHP - 1, 1)
        fire_scatter(1)
        return carry

    lax.fori_loop(0, 2, phase, 0, unroll=False)
    drain_scatter(0)
    drain_scatter(1)
    plsc.subcore_barrier()
    pltpu.sync_copy(shacc.at[pl.ds(s * 640, 640)],
                    acc_h.at[pl.ds(c * NP + s * 640, 640)])


def _pass2(xl_flat, srcB, dst, w):
    mesh = plsc.VectorSubcoreMesh(core_axis_name="c", subcore_axis_name="s")
    f = functools.partial(
        pl.kernel,
        out_type=jax.ShapeDtypeStruct((2 * NP, HH), jnp.float32),
        mesh=mesh,
        scratch_types=[
            pltpu.VMEM_SHARED((NP, HH), jnp.float32),
            pltpu.VMEM((EW2H,), jnp.int32),
            pltpu.VMEM((C2,), jnp.int32),
            pltpu.VMEM((C2,), jnp.int32),
            pltpu.VMEM((C2,), jnp.int32),
            pltpu.VMEM((C2,), jnp.int32),
            pltpu.VMEM((C2,), jnp.float32),
            pltpu.VMEM((C2,), jnp.float32),
            pltpu.VMEM((C2, HH), jnp.float32),
            pltpu.VMEM((C2, HH), jnp.float32),
            pltpu.VMEM((C2, HH), jnp.float32),
            pltpu.VMEM((C2, HH), jnp.float32),
            pltpu.SemaphoreType.DMA,
            pltpu.SemaphoreType.DMA,
            pltpu.SemaphoreType.DMA,
            pltpu.SemaphoreType.DMA,
        ],
    )(_pass2_body)
    return f(xl_flat, srcB, dst, w)


def _pool_body(accA_h, accB_h, den_h, selfw_h, xlA_h, xlB_h,
               bias_h, batch_h,
               psum_h, pmax_h, pcnt_h,
               biasv, batv, swv, d0v, bA, bB, bxA, bxB,
               gsum, gmax, gcnt):
    c = lax.axis_index("c")
    s = lax.axis_index("s")
    wid = s * 2 + c
    lane_iota = lax.iota(jnp.int32, 16)
    lane0 = jnp.where(lane_iota == 0, 1.0, 0.0)
    pltpu.sync_copy(bias_h, biasv)

    def zrow(g, carry):
        for j in range(16):
            sl = pl.ds(j * 16, 16)
            gsum[g, sl] = jnp.zeros((16,), jnp.float32)
            gmax[g, sl] = jnp.full((16,), NEG, jnp.float32)
        gcnt[g, pl.ds(0, 16)] = jnp.zeros((16,), jnp.float32)
        return carry

    lax.fori_loop(0, G, zrow, 0, unroll=False)

    def chunkfn(t, carry):
        base = wid * RW + t * PC
        pltpu.sync_copy(accA_h.at[pl.ds(base, PC)], bA)
        pltpu.sync_copy(accB_h.at[pl.ds(base, PC)], bB)
        pltpu.sync_copy(den_h.at[pl.ds(base, PC)], d0v)
        pltpu.sync_copy(selfw_h.at[pl.ds(base, PC)], swv)
        pltpu.sync_copy(xlA_h.at[pl.ds(base, PC)], bxA)
        pltpu.sync_copy(xlB_h.at[pl.ds(base, PC)], bxB)
        pltpu.sync_copy(batch_h.at[pl.ds(base, PC)], batv)

        def rowgroup(gg, carry2):
            batg = batv[pl.ds(gg * 16, 16)]
            swg = swv[pl.ds(gg * 16, 16)]
            d0g = d0v[pl.ds(gg * 16, 16)]
            invg = 1.0 / (d0g + swg)
            for i in range(16):
                r = gg * 16 + i
                b = batg[i]
                sw = swg[i]
                inv = invg[i]

                @pl.when(b < G)
                def _():
                    for j in range(8):
                        sl = pl.ds(j * 16, 16)
                        slB = pl.ds(HH + j * 16, 16)
                        hA = jnp.maximum(
                            (bA[r, sl] + sw * bxA[r, sl]) * inv + biasv[sl], 0.0)
                        hB = jnp.maximum(
                            (bB[r, sl] + sw * bxB[r, sl]) * inv + biasv[slB], 0.0)
                        gsum[b, sl] = gsum[b, sl] + hA
                        gsum[b, slB] = gsum[b, slB] + hB
                        gmax[b, sl] = jnp.maximum(gmax[b, sl], hA)
                        gmax[b, slB] = jnp.maximum(gmax[b, slB], hB)
                    gcnt[b, pl.ds(0, 16)] = gcnt[b, pl.ds(0, 16)] + lane0

            return carry2

        lax.fori_loop(0, PC // 16, rowgroup, 0, unroll=False)
        return carry

    lax.fori_loop(0, RW // PC, chunkfn, 0, unroll=False)
    pltpu.sync_copy(gsum, psum_h.at[pl.ds(wid * G, G)])
    pltpu.sync_copy(gmax, pmax_h.at[pl.ds(wid * G, G)])
    pltpu.sync_copy(gcnt, pcnt_h.at[pl.ds(wid * G, G)])


def _pool(accA, accB, den, selfw, xlA, xlB, bias, batch_pad):
    mesh = plsc.VectorSubcoreMesh(core_axis_name="c", subcore_axis_name="s")
    f = functools.partial(
        pl.kernel,
        out_type=[
            jax.ShapeDtypeStruct((32 * G, H), jnp.float32),
            jax.ShapeDtypeStruct((32 * G, H), jnp.float32),
            jax.ShapeDtypeStruct((32 * G, 16), jnp.float32),
        ],
        mesh=mesh,
        scratch_types=[
            pltpu.VMEM((H,), jnp.float32),
            pltpu.VMEM((PC,), jnp.int32),
            pltpu.VMEM((PC,), jnp.float32),
            pltpu.VMEM((PC,), jnp.float32),
            pltpu.VMEM((PC, HH), jnp.float32),
            pltpu.VMEM((PC, HH), jnp.float32),
            pltpu.VMEM((PC, HH), jnp.float32),
            pltpu.VMEM((PC, HH), jnp.float32),
            pltpu.VMEM((G, H), jnp.float32),
            pltpu.VMEM((G, H), jnp.float32),
            pltpu.VMEM((G, 16), jnp.float32),
        ],
    )(_pool_body)
    return f(accA, accB, den, selfw, xlA, xlB, bias, batch_pad)


def _k9_body(psum_ref, pmax_ref, pcnt_ref, fcW_ref, fcb_ref, out_ref):
    gsum = jnp.sum(psum_ref[...], axis=0)
    gmax = jnp.max(pmax_ref[...], axis=0)
    counts = jnp.sum(pcnt_ref[...], axis=(0, 2))
    gmean = gsum / jnp.maximum(counts, 1.0)[:, None]
    gmaxz = jnp.where(counts[:, None] > 0, gmax, 0.0)
    fcW = fcW_ref[...]
    out = (jnp.dot(gmaxz, fcW[:H, :], preferred_element_type=jnp.float32)
           + jnp.dot(gmean, fcW[H:, :], preferred_element_type=jnp.float32)
           + fcb_ref[0])
    out_ref[...] = out


def _k9(psum, pmax, pcnt, fcW, fcb):
    return pl.pallas_call(
        _k9_body,
        out_shape=jax.ShapeDtypeStruct((G, 1), jnp.float32),
    )(psum, pmax, pcnt, fcW, fcb)


def kernel(x, edge_attr, W1l, b1l, W1r, b1r, We1, att1, bias1,
           W2l, b2l, W2r, b2r, We2, att2, bias2, fcW, fcb,
           edge_index, batch):
    x_pad = jnp.pad(x, ((0, NP - N), (0, 0)))
    src = jnp.concatenate([edge_index[0], jnp.full((EP - E,), N, jnp.int32)])
    dst = jnp.concatenate([edge_index[1], jnp.full((EP - E,), N, jnp.int32)])
    ea0 = jnp.concatenate([edge_attr[:, 0], jnp.zeros((EP - E,), jnp.float32)])
    ea1 = jnp.concatenate([edge_attr[:, 1], jnp.zeros((EP - E,), jnp.float32)])
    ea_rs = jnp.reshape(edge_attr, (5000, 128))
    batch_pad = jnp.concatenate([batch, jnp.full((NP - N,), G, jnp.int32)])

    axl1, axr1, xlA1, xlB1, selfw1, mea = _k1(
        x_pad, W1l, b1l, W1r, b1r, We1, att1, ea_rs)
    w1, denp1 = _pass1(axl1, axr1, att1, We1, src, dst, ea0, ea1)
    denp1 = jnp.reshape(denp1, (2, NP))
    srcB = jnp.concatenate([src, src + NP])
    xl1_flat = jnp.concatenate([xlA1, xlB1], axis=0)
    acc1 = _pass2(xl1_flat, srcB, dst, w1)

    axl2, axr2, xlA2, xlB2, selfw2 = _k4(
        acc1[:NP], acc1[NP:], denp1, selfw1, xlA1, xlB1, bias1,
        W2l, b2l, W2r, b2r, We2, att2, mea)
    w2, denp2 = _pass1(axl2, axr2, att2, We2, src, dst, ea0, ea1)
    denp2 = jnp.reshape(denp2, (2, NP))
    xl2_flat = jnp.concatenate([xlA2, xlB2], axis=0)
    acc2 = _pass2(xl2_flat, srcB, dst, w2)
    den2 = _kden(denp2)

    psum, pmax, pcnt = _pool(
        acc2[:NP], acc2[NP:], den2, selfw2, xlA2, xlB2,
        bias2, batch_pad)
    psum = jnp.reshape(psum, (32, G, H))
    pmax = jnp.reshape(pmax, (32, G, H))
    pcnt = jnp.reshape(pcnt, (32, G, 16))
    return _k9(psum, pmax, pcnt, fcW, fcb)
